# Initial kernel scaffold; baseline (speedup 1.0000x reference)
import jax
import jax.numpy as jnp
from jax.experimental import pallas as pl


def kernel(xyz, feature, xyz_anchor):
    raise NotImplementedError("write your pallas kernel here")


# speedup vs baseline: 5.5100x; 5.5100x over previous
import jax
import jax.numpy as jnp
from jax.experimental import pallas as pl


def _knn_body(anchor_ref, xyz_ref, feat_ref, out_ref):
    a = anchor_ref[0]
    x = xyz_ref[0]
    mb = a.shape[1]
    n = x.shape[1]

    a2 = jnp.sum(a * a, axis=0)
    x2 = jnp.sum(x * x, axis=0)
    cross = jax.lax.dot_general(
        a, x, (((0,), (0,)), ((), ())), preferred_element_type=jnp.float32
    )
    dists = a2[:, None] + x2[None, :] - 2.0 * cross

    minval = jnp.min(dists, axis=1, keepdims=True)
    iota_mn = jax.lax.broadcasted_iota(jnp.int32, (mb, n), 1)
    idx = jnp.min(jnp.where(dists == minval, iota_mn, n), axis=1)

    onehot = (iota_mn == idx[:, None]).astype(jnp.float32)
    grouped = jax.lax.dot_general(
        x, onehot, (((1,), (1,)), ((), ())), preferred_element_type=jnp.float32
    )
    match = jnp.all(grouped == a, axis=0)

    iota_nm = jax.lax.broadcasted_iota(jnp.int32, (n, mb), 0)
    sel = ((iota_nm == idx[None, :]) & match[None, :]).astype(jnp.float32)
    feat = feat_ref[0]
    out_ref[0] = jax.lax.dot_general(
        feat, sel, (((1,), (0,)), ((), ())), preferred_element_type=jnp.float32
    )


def kernel(xyz, feature, xyz_anchor):
    B, C, N = feature.shape
    M = xyz_anchor.shape[2]
    MB = 512
    nmb = M // MB

    feature_anchor = pl.pallas_call(
        _knn_body,
        grid=(B, nmb),
        in_specs=[
            pl.BlockSpec((1, 3, MB), lambda b, m: (b, 0, m)),
            pl.BlockSpec((1, 3, N), lambda b, m: (b, 0, 0)),
            pl.BlockSpec((1, C, N), lambda b, m: (b, 0, 0)),
        ],
        out_specs=pl.BlockSpec((1, C, MB), lambda b, m: (b, 0, m)),
        out_shape=jax.ShapeDtypeStruct((B, C, M), jnp.float32),
    )(xyz_anchor, xyz, feature)

    return (xyz_anchor, feature_anchor)
